# baseline (speedup 1.0000x reference)
import functools

import jax
import jax.numpy as jnp
from jax import lax
from jax.experimental import pallas as pl
from jax.experimental.pallas import tpu as pltpu
from jax.experimental.pallas import tpu_sc as plsc

D = 32

_info = plsc.get_sparse_core_info()
_NC, _NS = _info.num_cores, _info.num_subcores
_NW = _NC * _NS

_G = 4
_NRING = 8


def _make_gather(n_samples: int, seq: int):
    per_w = n_samples // _NW
    ngroup = per_w // _G
    nsuper = ngroup // _NRING
    assert nsuper * _NRING * _G == per_w
    mesh = plsc.VectorSubcoreMesh(core_axis_name="c", subcore_axis_name="s")

    seq_p, d_p = 56, 128

    @functools.partial(
        pl.kernel,
        out_type=jax.ShapeDtypeStruct((n_samples, seq_p, d_p), jnp.float32),
        mesh=mesh,
        scratch_types=[
            pltpu.VMEM((per_w, seq), jnp.int32),
            [pltpu.VMEM((_G, seq, D), jnp.float32) for _ in range(_NRING)],
            pltpu.SemaphoreType.DMA((_NRING,)),
            pltpu.SemaphoreType.DMA((_NRING,)),
        ],
        compiler_params=pltpu.CompilerParams(use_tc_tiling_on_sc=False),
    )
    def gather_kernel(idx_hbm, table_hbm, out_hbm, idx_v, rows, in_sem, out_sem):
        wid = lax.axis_index("s") * _NC + lax.axis_index("c")
        r0 = wid * per_w
        pltpu.sync_copy(idx_hbm.at[pl.ds(r0, per_w)], idx_v)

        def fire_group(g, b):
            for k in range(_G):
                pltpu.async_copy(
                    table_hbm.at[idx_v.at[g * _G + k]],
                    rows[b].at[k],
                    in_sem.at[b],
                )

        def drain_group(b):
            for k in range(_G):
                pltpu.make_async_copy(
                    table_hbm.at[idx_v.at[k]],
                    rows[b].at[k],
                    in_sem.at[b],
                ).wait()

        def out_slice(g):
            return out_hbm.at[pl.ds(r0 + g * _G, _G), pl.ds(0, seq), pl.ds(0, D)]

        def wait_out(b):
            pltpu.make_async_copy(rows[b], out_slice(0), out_sem.at[b]).wait()

        def body(sg, carry):
            for b in range(_NRING):
                g = sg * _NRING + b
                drain_group(b)
                pltpu.async_copy(rows[b], out_slice(g), out_sem.at[b])

            @pl.when(sg + 1 < nsuper)
            def _():
                for b in range(_NRING):
                    g = (sg + 1) * _NRING + b
                    wait_out(b)
                    fire_group(g, b)

            return carry

        for b in range(_NRING):
            fire_group(b, b)
        lax.fori_loop(0, nsuper, body, 0)
        for b in range(_NRING):
            wait_out(b)

    return gather_kernel


def kernel(input, embeddings):
    n_samples, seq = input.shape
    out_padded = _make_gather(n_samples, seq)(input.astype(jnp.int32), embeddings)
    return out_padded[:, :seq, :D]


# speedup vs baseline: 1.0037x; 1.0037x over previous
import functools

import jax
import jax.numpy as jnp
from jax import lax
from jax.experimental import pallas as pl
from jax.experimental.pallas import tpu as pltpu
from jax.experimental.pallas import tpu_sc as plsc

D = 32

_info = plsc.get_sparse_core_info()
_NC, _NS = _info.num_cores, _info.num_subcores
_NW = _NC * _NS

_G = 8
_NRING = 4


def _make_gather(n_samples: int, seq: int):
    per_w = n_samples // _NW
    ngroup = per_w // _G
    nsuper = ngroup // _NRING
    assert nsuper * _NRING * _G == per_w
    mesh = plsc.VectorSubcoreMesh(core_axis_name="c", subcore_axis_name="s")

    seq_p, d_p = 56, 128

    @functools.partial(
        pl.kernel,
        out_type=jax.ShapeDtypeStruct((n_samples, seq_p, d_p), jnp.float32),
        mesh=mesh,
        scratch_types=[
            pltpu.VMEM((per_w, seq), jnp.int32),
            [pltpu.VMEM((_G, seq, D), jnp.float32) for _ in range(_NRING)],
            pltpu.SemaphoreType.DMA((_NRING,)),
            pltpu.SemaphoreType.DMA((_NRING,)),
        ],
        compiler_params=pltpu.CompilerParams(use_tc_tiling_on_sc=False),
    )
    def gather_kernel(idx_hbm, table_hbm, out_hbm, idx_v, rows, in_sem, out_sem):
        wid = lax.axis_index("s") * _NC + lax.axis_index("c")
        r0 = wid * per_w
        pltpu.sync_copy(idx_hbm.at[pl.ds(r0, per_w)], idx_v)

        def fire_group(g, b):
            for k in range(_G):
                pltpu.async_copy(
                    table_hbm.at[idx_v.at[g * _G + k]],
                    rows[b].at[k],
                    in_sem.at[b],
                )

        def drain_group(b):
            for k in range(_G):
                pltpu.make_async_copy(
                    table_hbm.at[idx_v.at[k]],
                    rows[b].at[k],
                    in_sem.at[b],
                ).wait()

        def out_slice(g):
            return out_hbm.at[pl.ds(r0 + g * _G, _G), pl.ds(0, seq), pl.ds(0, D)]

        def wait_out(b):
            pltpu.make_async_copy(rows[b], out_slice(0), out_sem.at[b]).wait()

        def body(sg, carry):
            for b in range(_NRING):
                g = sg * _NRING + b
                drain_group(b)
                pltpu.async_copy(rows[b], out_slice(g), out_sem.at[b])

            @pl.when(sg + 1 < nsuper)
            def _():
                for b in range(_NRING):
                    g = (sg + 1) * _NRING + b
                    wait_out(b)
                    fire_group(g, b)

            return carry

        for b in range(_NRING):
            fire_group(b, b)
        lax.fori_loop(0, nsuper, body, 0)
        for b in range(_NRING):
            wait_out(b)

    return gather_kernel


def kernel(input, embeddings):
    n_samples, seq = input.shape
    out_padded = _make_gather(n_samples, seq)(input.astype(jnp.int32), embeddings)
    return out_padded[:, :seq, :D]
